# baseline (speedup 1.0000x reference)
import functools

import jax
import jax.numpy as jnp
from jax import lax
from jax.experimental import pallas as pl
from jax.experimental.pallas import tpu as pltpu
from jax.experimental.pallas import tpu_sc as plsc

N = 10000
D = 768
C = 16
K0 = 3 * D

BM = 1000
NM = N // BM

NCOL1 = 13 * 128
NCOL2 = 7 * 128


def _stats_body(x_ref, sum_ref, sq_ref):
    @pl.when(pl.program_id(0) == 0)
    def _():
        sum_ref[...] = jnp.zeros_like(sum_ref)
        sq_ref[...] = jnp.zeros_like(sq_ref)
    xb = x_ref[...]
    s = jnp.sum(xb, axis=0, keepdims=True)
    q = jnp.sum(xb * xb, axis=0, keepdims=True)
    sum_ref[...] += jnp.broadcast_to(s, sum_ref.shape)
    sq_ref[...] += jnp.broadcast_to(q, sq_ref.shape)


def _bn_stats(x):
    out = pl.pallas_call(
        _stats_body,
        grid=(NM,),
        in_specs=[pl.BlockSpec((BM, K0), lambda m: (m, 0))],
        out_specs=[pl.BlockSpec((8, K0), lambda m: (0, 0)),
                   pl.BlockSpec((8, K0), lambda m: (0, 0))],
        out_shape=[jax.ShapeDtypeStruct((8, K0), jnp.float32),
                   jax.ShapeDtypeStruct((8, K0), jnp.float32)],
    )(x)
    return out[0][0], out[1][0]


def _p1_body(x_ref, s_ref, w_ref, o1_ref, o2_ref, o3_ref):
    xb = (x_ref[...] * s_ref[0:1, :]).astype(jnp.bfloat16)
    r = jnp.dot(xb, w_ref[...], preferred_element_type=jnp.float32)
    o1_ref[...] = r[:, :768]
    o2_ref[...] = r[:, 768:1536]
    o3_ref[...] = r[:, 1536:]


def _p1(x, scale8, wbig_bf):
    return pl.pallas_call(
        _p1_body,
        grid=(NM,),
        in_specs=[pl.BlockSpec((BM, K0), lambda m: (m, 0)),
                  pl.BlockSpec((8, K0), lambda m: (0, 0)),
                  pl.BlockSpec((K0, NCOL1), lambda m: (0, 0))],
        out_specs=[pl.BlockSpec((BM, 768), lambda m: (m, 0)),
                   pl.BlockSpec((BM, 768), lambda m: (m, 0)),
                   pl.BlockSpec((BM, 128), lambda m: (m, 0))],
        out_shape=[jax.ShapeDtypeStruct((N, 768), jnp.float32),
                   jax.ShapeDtypeStruct((N, 768), jnp.float32),
                   jax.ShapeDtypeStruct((N, 128), jnp.float32)],
    )(x, scale8, wbig_bf)


def _p2_body(h1_ref, xc2_ref, w_ref, oh_ref, oe_ref):
    hb = h1_ref[...].astype(jnp.bfloat16)
    r = jnp.dot(hb, w_ref[...], preferred_element_type=jnp.float32)
    oh_ref[...] = r[:, :768] + xc2_ref[...]
    oe_ref[...] = r[:, 768:]


def _p2(h1, xc2, w2cat_bf):
    return pl.pallas_call(
        _p2_body,
        grid=(NM,),
        in_specs=[pl.BlockSpec((BM, 768), lambda m: (m, 0)),
                  pl.BlockSpec((BM, 768), lambda m: (m, 0)),
                  pl.BlockSpec((768, NCOL2), lambda m: (0, 0))],
        out_specs=[pl.BlockSpec((BM, 768), lambda m: (m, 0)),
                   pl.BlockSpec((BM, 128), lambda m: (m, 0))],
        out_shape=[jax.ShapeDtypeStruct((N, 768), jnp.float32),
                   jax.ShapeDtypeStruct((N, 128), jnp.float32)],
    )(h1, xc2, w2cat_bf)


def _loss_body(h2_ref, o3_ref, tgt_ref, pw_ref, cpool_ref, cdir_ref,
               pool_ref, part_ref):
    h2b = h2_ref[...].astype(jnp.bfloat16)
    pool_logits = jnp.dot(h2b, pw_ref[...], preferred_element_type=jnp.float32)
    pool_logits = pool_logits + cpool_ref[0:1, :]
    dir_logits = o3_ref[...] + cdir_ref[0:1, :]

    lanes = lax.broadcasted_iota(jnp.int32, (BM, 128), 1)
    maskC = lanes < C
    t = tgt_ref[0, 0, :].reshape(BM, 1)
    valid = (t >= 0)
    t_safe = jnp.where(valid, t, 0)
    sel_mask = (lanes == t_safe)

    def ce_rows(L):
        Lm = jnp.where(maskC, L, -jnp.inf)
        m = jnp.max(Lm, axis=1, keepdims=True)
        lse = m + jnp.log(jnp.sum(jnp.where(maskC, jnp.exp(L - m), 0.0),
                                  axis=1, keepdims=True))
        sel = jnp.sum(jnp.where(sel_mask, L, 0.0), axis=1, keepdims=True)
        return lse - sel, Lm, m

    nllp, Lp, mp = ce_rows(pool_logits)
    nlld, _, _ = ce_rows(dir_logits)
    ismax = (Lp == mp) & maskC
    pred = jnp.min(jnp.where(ismax, lanes, 128), axis=1, keepdims=True)
    correct = ((pred == t) & valid).astype(jnp.float32)
    validf = valid.astype(jnp.float32)

    contrib = (nllp * validf * (lanes == 0) + nlld * validf * (lanes == 1)
               + validf * (lanes == 2) + correct * (lanes == 3))
    part = jnp.sum(contrib, axis=0, keepdims=True)
    part_ref[...] = jnp.broadcast_to(part, part_ref.shape)
    pool_ref[...] = pool_logits


def _loss(h2, o3, tgt_r, pwpad_bf, cpool8, cdir8):
    return pl.pallas_call(
        _loss_body,
        grid=(NM,),
        in_specs=[pl.BlockSpec((BM, 768), lambda m: (m, 0)),
                  pl.BlockSpec((BM, 128), lambda m: (m, 0)),
                  pl.BlockSpec((1, 1, BM), lambda m: (m, 0, 0)),
                  pl.BlockSpec((768, 128), lambda m: (0, 0)),
                  pl.BlockSpec((8, 128), lambda m: (0, 0)),
                  pl.BlockSpec((8, 128), lambda m: (0, 0))],
        out_specs=[pl.BlockSpec((BM, 128), lambda m: (m, 0)),
                   pl.BlockSpec((1, 8, 128), lambda m: (m, 0, 0))],
        out_shape=[jax.ShapeDtypeStruct((N, 128), jnp.float32),
                   jax.ShapeDtypeStruct((NM, 8, 128), jnp.float32)],
    )(h2, o3, tgt_r, pwpad_bf, cpool8, cdir8)


E_TOT = 160000 + N
NS = 16
NCORE = 2
G = 64
NG = 168
EPT = NG * G
E_PAD = NS * EPT
NP = 10240
STRIPE = NP // NS
SUB = 40
CHUNKS = 6
CPC = CHUNKS // NCORE
SB = 768
NSB = EPT // SB
GPS = SB // G


def _splat_i32(v):
    return jnp.zeros((16,), jnp.int32) + v


def _gat_sc_body(hflat, asrc, adst, sidxf, didxf, brow, out,
                 sidx_sb, didx_sb, fidx_sb, as_g, ad_g, exloc, inv_full,
                 zbuf, gbufA, gbufB, wbuf, biasv,
                 acc_sh, den_sh, inv_sh,
                 semA, semB, semSA, semSB, semP):
    c = lax.axis_index("c")
    s = lax.axis_index("s")
    iota16 = lax.broadcasted_iota(jnp.int32, (16,), 0)
    r0 = s * STRIPE

    def _z16(i, carry):
        zbuf[pl.ds(i * 16, 16)] = jnp.zeros((16,), jnp.float32)
        return carry
    lax.fori_loop(0, STRIPE // 16, _z16, None)
    pltpu.sync_copy(zbuf, den_sh.at[pl.ds(r0, STRIPE)])
    plsc.subcore_barrier()

    def _p1(sb, carry):
        base = sb * SB
        pltpu.sync_copy(sidxf.at[s, pl.ds(base, SB)], sidx_sb)
        pltpu.sync_copy(didxf.at[s, pl.ds(base, SB)], didx_sb)
        for q in range(GPS):
            sl = pl.ds(q * G, G)
            pltpu.async_copy(asrc.at[sidx_sb.at[sl]], as_g.at[sl], semP)
            pltpu.async_copy(adst.at[didx_sb.at[sl]], ad_g.at[sl], semP)
        for q in range(GPS):
            sl = pl.ds(q * G, G)
            pltpu.make_async_copy(asrc.at[sidx_sb.at[sl]], as_g.at[sl], semP).wait()
            pltpu.make_async_copy(adst.at[didx_sb.at[sl]], ad_g.at[sl], semP).wait()

        def _ex16(i, carry2):
            off = i * 16
            sl = pl.ds(off, 16)
            al = as_g[sl] + ad_g[sl]
            al = jnp.where(al >= 0, al, 0.2 * al)
            ex = jnp.exp(al)
            gid = s * EPT + base + off + iota16
            ex = jnp.where(gid < E_TOT, ex, 0.0)
            exloc[pl.ds(base + off, 16)] = ex
            return carry2
        lax.fori_loop(0, SB // 16, _ex16, None)

        for b in range(SB // 16 // 8):
            idxs = [didx_sb[pl.ds((b * 8 + q) * 16, 16)] for q in range(8)]
            for q in range(8):
                pltpu.async_copy(exloc.at[pl.ds(base + (b * 8 + q) * 16, 16)],
                                 den_sh.at[idxs[q]], semP, add=True)
            for q in range(8):
                pltpu.make_async_copy(exloc.at[pl.ds(base + (b * 8 + q) * 16, 16)],
                                      den_sh.at[idxs[q]], semP).wait()
        return carry
    lax.fori_loop(0, NSB, _p1, None)
    plsc.subcore_barrier()

    pltpu.sync_copy(den_sh.at[pl.ds(r0, STRIPE)], zbuf)

    def _inv16(i, carry):
        sl = pl.ds(i * 16, 16)
        zbuf[sl] = 1.0 / (zbuf[sl] + 1e-16)
        return carry
    lax.fori_loop(0, STRIPE // 16, _inv16, None)
    pltpu.sync_copy(zbuf, inv_sh.at[pl.ds(r0, STRIPE)])
    plsc.subcore_barrier()
    pltpu.sync_copy(inv_sh, inv_full)

    def _p1d(sb, carry):
        base = sb * SB
        pltpu.sync_copy(didxf.at[s, pl.ds(base, SB)], didx_sb)

        def _c16(i, carry2):
            off = i * 16
            iv = plsc.load_gather(inv_full, [didx_sb[pl.ds(off, 16)]])
            sl = pl.ds(base + off, 16)
            exloc[sl] = exloc[sl] * iv
            return carry2
        lax.fori_loop(0, SB // 16, _c16, None)
        return carry
    lax.fori_loop(0, NSB, _p1d, None)

    def _gidx(g):
        return hflat.at[fidx_sb.at[pl.ds(g * G, G)]]

    def _scat(buf, g, sem, start):
        for h in range(G // 16):
            ih = didx_sb[pl.ds(g * G + h * 16, 16)]
            if start:
                pltpu.async_copy(buf.at[pl.ds(h * 16, 16)], acc_sh.at[ih],
                                 sem, add=True)
            else:
                pltpu.make_async_copy(buf.at[pl.ds(h * 16, 16)], acc_sh.at[ih],
                                      sem).wait()

    def _scale(buf, basew):
        for r in range(G):
            cf = plsc.load_gather(exloc, [_splat_i32(basew + r)])
            for l in range(8):
                sl = pl.ds(l * 16, 16)
                buf[r, sl] = buf[r, sl] * cf

    def _chunk(j, carry0):
        jc = c * CPC + j
        pltpu.sync_copy(brow.at[pl.ds(jc * 128, 128)], biasv)
        bvals = [biasv[pl.ds(l * 16, 16)] for l in range(8)]
        for r in range(SUB):
            for l in range(8):
                wbuf[r, pl.ds(l * 16, 16)] = bvals[l]
        for k in range(STRIPE // SUB):
            pltpu.sync_copy(wbuf, acc_sh.at[pl.ds(r0 + k * SUB, SUB)])
        plsc.subcore_barrier()

        def _sb2(sb, carry):
            base = sb * SB
            pltpu.sync_copy(sidxf.at[s, pl.ds(base, SB)], sidx_sb)
            pltpu.sync_copy(didxf.at[s, pl.ds(base, SB)], didx_sb)

            def _fx(i, carry2):
                sl = pl.ds(i * 16, 16)
                fidx_sb[sl] = sidx_sb[sl] * CHUNKS + jc
                return carry2
            lax.fori_loop(0, SB // 16, _fx, None)

            pltpu.async_copy(_gidx(0), gbufA, semA)

            def _pair(p, carry2):
                g0 = 2 * p
                g1 = 2 * p + 1
                pltpu.make_async_copy(_gidx(g0), gbufA, semA).wait()

                @pl.when(p > 0)
                def _():
                    _scat(gbufB, g1 - 2, semSB, False)
                pltpu.async_copy(_gidx(g1), gbufB, semB)
                _scale(gbufA, base + g0 * G)
                _scat(gbufA, g0, semSA, True)
                pltpu.make_async_copy(_gidx(g1), gbufB, semB).wait()
                _scat(gbufA, g0, semSA, False)

                @pl.when(p < GPS // 2 - 1)
                def _():
                    pltpu.async_copy(_gidx(g0 + 2), gbufA, semA)
                _scale(gbufB, base + g1 * G)
                _scat(gbufB, g1, semSB, True)
                return carry2
            lax.fori_loop(0, GPS // 2, _pair, None)
            _scat(gbufB, GPS - 1, semSB, False)
            return carry
        lax.fori_loop(0, NSB, _sb2, None)
        plsc.subcore_barrier()

        for k in range(STRIPE // SUB):
            rk = r0 + k * SUB

            @pl.when(rk < N)
            def _():
                pltpu.sync_copy(acc_sh.at[pl.ds(rk, SUB)], wbuf)
                pltpu.sync_copy(wbuf, out.at[pl.ds(rk, SUB), pl.ds(jc * 128, 128)])
        plsc.subcore_barrier()
        return carry0
    lax.fori_loop(0, CPC, _chunk, None)


def _gat_sc(hflat, asrc, adst, sidxf, didxf, brow):
    mesh = plsc.VectorSubcoreMesh(core_axis_name="c", subcore_axis_name="s")
    f = pl.kernel(
        _gat_sc_body,
        out_type=jax.ShapeDtypeStruct((N, 768), jnp.float32),
        mesh=mesh,
        compiler_params=pltpu.CompilerParams(needs_layout_passes=False),
        scratch_types=[
            pltpu.VMEM((SB,), jnp.int32),
            pltpu.VMEM((SB,), jnp.int32),
            pltpu.VMEM((SB,), jnp.int32),
            pltpu.VMEM((SB,), jnp.float32),
            pltpu.VMEM((SB,), jnp.float32),
            pltpu.VMEM((EPT,), jnp.float32),
            pltpu.VMEM((NP,), jnp.float32),
            pltpu.VMEM((STRIPE,), jnp.float32),
            pltpu.VMEM((G, 128), jnp.float32),
            pltpu.VMEM((G, 128), jnp.float32),
            pltpu.VMEM((SUB, 128), jnp.float32),
            pltpu.VMEM((128,), jnp.float32),
            pltpu.VMEM_SHARED((NP, 128), jnp.float32),
            pltpu.VMEM_SHARED((NP,), jnp.float32),
            pltpu.VMEM_SHARED((NP,), jnp.float32),
            pltpu.SemaphoreType.DMA,
            pltpu.SemaphoreType.DMA,
            pltpu.SemaphoreType.DMA,
            pltpu.SemaphoreType.DMA,
            pltpu.SemaphoreType.DMA,
        ],
    )
    return f(hflat, asrc, adst, sidxf, didxf, brow)


def _edge_phase(hraw, asrc, adst, bias_row, sidxf, didxf):
    return _gat_sc(hraw.reshape(N * CHUNKS, 128), asrc, adst,
                   sidxf, didxf, bias_row)


def kernel(x, edge_index, target, bn_gamma, bn_beta, W1, att_src1, att_dst1,
           b1, W2, att_src2, att_dst2, b2, pool_W, pool_b, dir_W, dir_b):
    src, dst = edge_index[0], edge_index[1]
    loops = jnp.arange(N, dtype=src.dtype)
    pad = jnp.zeros((E_PAD - E_TOT,), src.dtype)
    s_pad = jnp.concatenate([src, loops, pad])
    d_pad = jnp.concatenate([dst, loops, pad])
    sidxf = s_pad.reshape(NS, EPT)
    didxf = d_pad.reshape(NS, EPT)

    W2a, W2b = W2[:D], W2[D:]

    w1as = W1 @ att_src1
    w1ad = W1 @ att_dst1
    w2bs = W2b @ att_src2
    w2bd = W2b @ att_dst2
    extras1 = jnp.zeros((K0, 128), jnp.float32)
    extras1 = extras1.at[:, 0:16].set(dir_W)
    extras1 = extras1.at[:, 16].set(w1as)
    extras1 = extras1.at[:, 17].set(w1ad)
    extras1 = extras1.at[:, 18].set(w2bs)
    extras1 = extras1.at[:, 19].set(w2bd)
    wbig_bf = jnp.concatenate([W1, W2b, extras1], axis=1).astype(jnp.bfloat16)

    w2as = W2a @ att_src2
    w2ad = W2a @ att_dst2
    extras2 = jnp.zeros((D, 128), jnp.float32)
    extras2 = extras2.at[:, 0].set(w2as)
    extras2 = extras2.at[:, 1].set(w2ad)
    w2cat_bf = jnp.concatenate([W2a, extras2], axis=1).astype(jnp.bfloat16)

    pwpad = jnp.zeros((D, 128), jnp.float32).at[:, :C].set(pool_W)
    pwpad_bf = pwpad.astype(jnp.bfloat16)

    sums, sqs = _bn_stats(x)
    mean = sums / N
    var = sqs / N - mean * mean
    scale = bn_gamma / jnp.sqrt(var + 1e-5)
    shift = bn_beta - mean * scale
    scale8 = jnp.broadcast_to(scale[None, :], (8, K0))

    r1 = shift @ W1
    r2 = shift @ W2b

    o1, o2, o3 = _p1(x, scale8, wbig_bf)

    asrc1 = o3[:, 16] + jnp.dot(shift, w1as)
    adst1 = o3[:, 17] + jnp.dot(shift, w1ad)

    h1 = _edge_phase(o1, asrc1, adst1, r1 + b1, sidxf, didxf)

    p2h, p2e = _p2(h1, o2, w2cat_bf)
    asrc2 = o3[:, 18] + p2e[:, 0] + jnp.dot(r2, att_src2)
    adst2 = o3[:, 19] + p2e[:, 1] + jnp.dot(r2, att_dst2)

    h2 = _edge_phase(p2h, asrc2, adst2, r2 + b2, sidxf, didxf)

    cdir = jnp.zeros((128,), jnp.float32).at[:C].set(shift @ dir_W + dir_b)
    cpool = jnp.zeros((128,), jnp.float32).at[:C].set(pool_b)
    cdir8 = jnp.broadcast_to(cdir[None, :], (8, 128))
    cpool8 = jnp.broadcast_to(cpool[None, :], (8, 128))
    tgt_r = target.reshape(NM, 1, BM)

    poolpad, partials = _loss(h2, o3, tgt_r, pwpad_bf, cpool8, cdir8)
    pooler = poolpad[:, :C]
    l = jnp.sum(partials[:, 0, :], axis=0)
    nv = jnp.maximum(l[2], 1.0)
    loss = (l[0] + l[1]) / nv
    acc = l[3] / nv
    return (h2, pooler, loss, acc)


# speedup vs baseline: 1.0848x; 1.0848x over previous
import functools

import jax
import jax.numpy as jnp
import numpy as np
from jax import lax
from jax.experimental import pallas as pl
from jax.experimental.pallas import tpu as pltpu
from jax.experimental.pallas import tpu_sc as plsc

N = 10000
D = 768
C = 16
K0 = 3 * D

BM = 1000
NM = N // BM

NCOL1 = 13 * 128
NCOL2 = 7 * 128


def _stats_body(x_ref, sum_ref, sq_ref):
    @pl.when(pl.program_id(0) == 0)
    def _():
        sum_ref[...] = jnp.zeros_like(sum_ref)
        sq_ref[...] = jnp.zeros_like(sq_ref)
    xb = x_ref[...]
    s = jnp.sum(xb, axis=0, keepdims=True)
    q = jnp.sum(xb * xb, axis=0, keepdims=True)
    sum_ref[...] += jnp.broadcast_to(s, sum_ref.shape)
    sq_ref[...] += jnp.broadcast_to(q, sq_ref.shape)


def _bn_stats(x):
    out = pl.pallas_call(
        _stats_body,
        grid=(NM,),
        in_specs=[pl.BlockSpec((BM, K0), lambda m: (m, 0))],
        out_specs=[pl.BlockSpec((8, K0), lambda m: (0, 0)),
                   pl.BlockSpec((8, K0), lambda m: (0, 0))],
        out_shape=[jax.ShapeDtypeStruct((8, K0), jnp.float32),
                   jax.ShapeDtypeStruct((8, K0), jnp.float32)],
    )(x)
    return out[0][0], out[1][0]


def _p1_body(x_ref, s_ref, w_ref, o1_ref, o2_ref, o3_ref):
    xb = (x_ref[...] * s_ref[0:1, :]).astype(jnp.bfloat16)
    r = jnp.dot(xb, w_ref[...], preferred_element_type=jnp.float32)
    o1_ref[...] = r[:, :768]
    o2_ref[...] = r[:, 768:1536]
    o3_ref[...] = r[:, 1536:]


def _p1(x, scale8, wbig_bf):
    return pl.pallas_call(
        _p1_body,
        grid=(NM,),
        in_specs=[pl.BlockSpec((BM, K0), lambda m: (m, 0)),
                  pl.BlockSpec((8, K0), lambda m: (0, 0)),
                  pl.BlockSpec((K0, NCOL1), lambda m: (0, 0))],
        out_specs=[pl.BlockSpec((BM, 768), lambda m: (m, 0)),
                   pl.BlockSpec((BM, 768), lambda m: (m, 0)),
                   pl.BlockSpec((BM, 128), lambda m: (m, 0))],
        out_shape=[jax.ShapeDtypeStruct((N, 768), jnp.float32),
                   jax.ShapeDtypeStruct((N, 768), jnp.float32),
                   jax.ShapeDtypeStruct((N, 128), jnp.float32)],
    )(x, scale8, wbig_bf)


def _p2_body(h1_ref, xc2_ref, w_ref, oh_ref, oe_ref):
    hb = h1_ref[...].astype(jnp.bfloat16)
    r = jnp.dot(hb, w_ref[...], preferred_element_type=jnp.float32)
    oh_ref[...] = r[:, :768] + xc2_ref[...]
    oe_ref[...] = r[:, 768:]


def _p2(h1, xc2, w2cat_bf):
    return pl.pallas_call(
        _p2_body,
        grid=(NM,),
        in_specs=[pl.BlockSpec((BM, 768), lambda m: (m, 0)),
                  pl.BlockSpec((BM, 768), lambda m: (m, 0)),
                  pl.BlockSpec((768, NCOL2), lambda m: (0, 0))],
        out_specs=[pl.BlockSpec((BM, 768), lambda m: (m, 0)),
                   pl.BlockSpec((BM, 128), lambda m: (m, 0))],
        out_shape=[jax.ShapeDtypeStruct((N, 768), jnp.float32),
                   jax.ShapeDtypeStruct((N, 128), jnp.float32)],
    )(h1, xc2, w2cat_bf)


def _loss_body(h2_ref, o3_ref, tgt_ref, pw_ref, cpool_ref, cdir_ref,
               pool_ref, part_ref):
    h2b = h2_ref[...].astype(jnp.bfloat16)
    pool_logits = jnp.dot(h2b, pw_ref[...], preferred_element_type=jnp.float32)
    pool_logits = pool_logits + cpool_ref[0:1, :]
    dir_logits = o3_ref[...] + cdir_ref[0:1, :]

    lanes = lax.broadcasted_iota(jnp.int32, (BM, 128), 1)
    maskC = lanes < C
    t = tgt_ref[0, 0, :].reshape(BM, 1)
    valid = (t >= 0)
    t_safe = jnp.where(valid, t, 0)
    sel_mask = (lanes == t_safe)

    def ce_rows(L):
        Lm = jnp.where(maskC, L, -jnp.inf)
        m = jnp.max(Lm, axis=1, keepdims=True)
        lse = m + jnp.log(jnp.sum(jnp.where(maskC, jnp.exp(L - m), 0.0),
                                  axis=1, keepdims=True))
        sel = jnp.sum(jnp.where(sel_mask, L, 0.0), axis=1, keepdims=True)
        return lse - sel, Lm, m

    nllp, Lp, mp = ce_rows(pool_logits)
    nlld, _, _ = ce_rows(dir_logits)
    ismax = (Lp == mp) & maskC
    pred = jnp.min(jnp.where(ismax, lanes, 128), axis=1, keepdims=True)
    correct = ((pred == t) & valid).astype(jnp.float32)
    validf = valid.astype(jnp.float32)

    contrib = (nllp * validf * (lanes == 0) + nlld * validf * (lanes == 1)
               + validf * (lanes == 2) + correct * (lanes == 3))
    part = jnp.sum(contrib, axis=0, keepdims=True)
    part_ref[...] = jnp.broadcast_to(part, part_ref.shape)
    pool_ref[...] = pool_logits


def _loss(h2, o3, tgt_r, pwpad_bf, cpool8, cdir8):
    return pl.pallas_call(
        _loss_body,
        grid=(NM,),
        in_specs=[pl.BlockSpec((BM, 768), lambda m: (m, 0)),
                  pl.BlockSpec((BM, 128), lambda m: (m, 0)),
                  pl.BlockSpec((1, 1, BM), lambda m: (m, 0, 0)),
                  pl.BlockSpec((768, 128), lambda m: (0, 0)),
                  pl.BlockSpec((8, 128), lambda m: (0, 0)),
                  pl.BlockSpec((8, 128), lambda m: (0, 0))],
        out_specs=[pl.BlockSpec((BM, 128), lambda m: (m, 0)),
                   pl.BlockSpec((1, 8, 128), lambda m: (m, 0, 0))],
        out_shape=[jax.ShapeDtypeStruct((N, 128), jnp.float32),
                   jax.ShapeDtypeStruct((NM, 8, 128), jnp.float32)],
    )(h2, o3, tgt_r, pwpad_bf, cpool8, cdir8)


E_TOT = 160000 + N
NS = 16
NCORE = 2
G = 16
NG = 672
NB = 8
EPT = NG * G
E_PAD = NS * EPT
NP = 10240
STRIPE = NP // NS
SUB = 40
CHUNKS = 6
CPC = CHUNKS // NCORE
SB = 768
NSB = EPT // SB
GPS = SB // G


def _splat_i32(v):
    return jnp.zeros((16,), jnp.int32) + v


def _gat_sc_body(hflat, asrc, adst, sidxf, didxf, brow, out,
                 sidx_sb, didx_sb, fidx_sb, as_g, ad_g, exloc, inv_full,
                 zbuf, b0, b1, b2, b3, b4, b5, b6, b7, wbuf, biasv,
                 acc_sh, den_sh, inv_sh,
                 m0, m1, m2, m3, m4, m5, m6, m7, semP):
    c = lax.axis_index("c")
    s = lax.axis_index("s")
    iota16 = lax.broadcasted_iota(jnp.int32, (16,), 0)
    r0 = s * STRIPE

    def _z16(i, carry):
        zbuf[pl.ds(i * 16, 16)] = jnp.zeros((16,), jnp.float32)
        return carry
    lax.fori_loop(0, STRIPE // 16, _z16, None)
    pltpu.sync_copy(zbuf, den_sh.at[pl.ds(r0, STRIPE)])
    plsc.subcore_barrier()

    def _p1(sb, carry):
        base = sb * SB
        pltpu.sync_copy(sidxf.at[s, pl.ds(base, SB)], sidx_sb)
        pltpu.sync_copy(didxf.at[s, pl.ds(base, SB)], didx_sb)
        for q in range(GPS):
            sl = pl.ds(q * G, G)
            pltpu.async_copy(asrc.at[sidx_sb.at[sl]], as_g.at[sl], semP)
            pltpu.async_copy(adst.at[didx_sb.at[sl]], ad_g.at[sl], semP)
        for q in range(GPS):
            sl = pl.ds(q * G, G)
            pltpu.make_async_copy(asrc.at[sidx_sb.at[sl]], as_g.at[sl], semP).wait()
            pltpu.make_async_copy(adst.at[didx_sb.at[sl]], ad_g.at[sl], semP).wait()

        def _ex16(i, carry2):
            off = i * 16
            sl = pl.ds(off, 16)
            al = as_g[sl] + ad_g[sl]
            al = jnp.where(al >= 0, al, 0.2 * al)
            ex = jnp.exp(al)
            gid = s * EPT + base + off + iota16
            ex = jnp.where(gid < E_TOT, ex, 0.0)
            exloc[pl.ds(base + off, 16)] = ex
            return carry2
        lax.fori_loop(0, SB // 16, _ex16, None)

        for b in range(SB // 16 // 8):
            idxs = [didx_sb[pl.ds((b * 8 + q) * 16, 16)] for q in range(8)]
            for q in range(8):
                pltpu.async_copy(exloc.at[pl.ds(base + (b * 8 + q) * 16, 16)],
                                 den_sh.at[idxs[q]], semP, add=True)
            for q in range(8):
                pltpu.make_async_copy(exloc.at[pl.ds(base + (b * 8 + q) * 16, 16)],
                                      den_sh.at[idxs[q]], semP).wait()
        return carry
    lax.fori_loop(0, NSB, _p1, None)
    plsc.subcore_barrier()

    pltpu.sync_copy(den_sh.at[pl.ds(r0, STRIPE)], zbuf)

    def _inv16(i, carry):
        sl = pl.ds(i * 16, 16)
        zbuf[sl] = 1.0 / (zbuf[sl] + 1e-16)
        return carry
    lax.fori_loop(0, STRIPE // 16, _inv16, None)
    pltpu.sync_copy(zbuf, inv_sh.at[pl.ds(r0, STRIPE)])
    plsc.subcore_barrier()
    pltpu.sync_copy(inv_sh, inv_full)

    def _p1d(sb, carry):
        base = sb * SB
        pltpu.sync_copy(didxf.at[s, pl.ds(base, SB)], didx_sb)

        def _c16(i, carry2):
            off = i * 16
            iv = plsc.load_gather(inv_full, [didx_sb[pl.ds(off, 16)]])
            sl = pl.ds(base + off, 16)
            exloc[sl] = exloc[sl] * iv
            return carry2
        lax.fori_loop(0, SB // 16, _c16, None)
        return carry
    lax.fori_loop(0, NSB, _p1d, None)

    def _gidx(g):
        return hflat.at[fidx_sb.at[pl.ds(g * G, G)]]

    def _scat(buf, g, sem, start):
        ih = didx_sb[pl.ds(g * G, 16)]
        if start:
            pltpu.async_copy(buf, acc_sh.at[ih], sem, add=True)
        else:
            pltpu.make_async_copy(buf, acc_sh.at[ih], sem).wait()

    def _scale(buf, basew):
        for r in range(G):
            cf = plsc.load_gather(exloc, [_splat_i32(basew + r)])
            for l in range(8):
                sl = pl.ds(l * 16, 16)
                buf[r, sl] = buf[r, sl] * cf

    def _chunk(j, carry0):
        jc = c * CPC + j
        pltpu.sync_copy(brow.at[pl.ds(jc * 128, 128)], biasv)
        bvals = [biasv[pl.ds(l * 16, 16)] for l in range(8)]
        for r in range(SUB):
            for l in range(8):
                wbuf[r, pl.ds(l * 16, 16)] = bvals[l]
        for k in range(STRIPE // SUB):
            pltpu.sync_copy(wbuf, acc_sh.at[pl.ds(r0 + k * SUB, SUB)])
        plsc.subcore_barrier()

        def _sb2(sb, carry):
            base = sb * SB
            pltpu.sync_copy(sidxf.at[s, pl.ds(base, SB)], sidx_sb)
            pltpu.sync_copy(didxf.at[s, pl.ds(base, SB)], didx_sb)

            def _fx(i, carry2):
                sl = pl.ds(i * 16, 16)
                fidx_sb[sl] = sidx_sb[sl] * CHUNKS + jc
                return carry2
            lax.fori_loop(0, SB // 16, _fx, None)

            bufs = (b0, b1, b2, b3, b4, b5, b6, b7)
            sems = (m0, m1, m2, m3, m4, m5, m6, m7)
            for q in range(NB - 2):
                pltpu.async_copy(_gidx(q), bufs[q], sems[q])

            def _ring(t, carry2):
                for u in range(NB):
                    g = NB * t + u
                    buf, sm = bufs[u], sems[u]
                    pltpu.make_async_copy(_gidx(g), buf, sm).wait()
                    _scale(buf, base + g * G)
                    _scat(buf, g, sm, True)
                    pbuf = bufs[(u + 6) % NB]
                    psm = sems[(u + 6) % NB]
                    if u < 2:
                        @pl.when(t > 0)
                        def _():
                            _scat(pbuf, g - 2, psm, False)
                        pltpu.async_copy(_gidx(g + 6), pbuf, psm)
                    else:
                        _scat(pbuf, g - 2, psm, False)

                        @pl.when(t < GPS // NB - 1)
                        def _():
                            pltpu.async_copy(_gidx(g + 6), pbuf, psm)
                return carry2
            lax.fori_loop(0, GPS // NB, _ring, None)
            _scat(bufs[(GPS - 2) % NB], GPS - 2, sems[(GPS - 2) % NB], False)
            _scat(bufs[(GPS - 1) % NB], GPS - 1, sems[(GPS - 1) % NB], False)
            return carry
        lax.fori_loop(0, NSB, _sb2, None)
        plsc.subcore_barrier()

        for k in range(STRIPE // SUB):
            rk = r0 + k * SUB

            @pl.when(rk < N)
            def _():
                pltpu.sync_copy(acc_sh.at[pl.ds(rk, SUB)], wbuf)
                pltpu.sync_copy(wbuf, out.at[pl.ds(rk, SUB), pl.ds(jc * 128, 128)])
        plsc.subcore_barrier()
        return carry0
    lax.fori_loop(0, CPC, _chunk, None)


def _gat_sc(hflat, asrc, adst, sidxf, didxf, brow):
    mesh = plsc.VectorSubcoreMesh(core_axis_name="c", subcore_axis_name="s")
    f = pl.kernel(
        _gat_sc_body,
        out_type=jax.ShapeDtypeStruct((N, 768), jnp.float32),
        mesh=mesh,
        name="gat_sc",
        compiler_params=pltpu.CompilerParams(needs_layout_passes=False),
        scratch_types=[
            pltpu.VMEM((SB,), jnp.int32),
            pltpu.VMEM((SB,), jnp.int32),
            pltpu.VMEM((SB,), jnp.int32),
            pltpu.VMEM((SB,), jnp.float32),
            pltpu.VMEM((SB,), jnp.float32),
            pltpu.VMEM((EPT,), jnp.float32),
            pltpu.VMEM((NP,), jnp.float32),
            pltpu.VMEM((STRIPE,), jnp.float32),
        ] + [pltpu.VMEM((G, 128), jnp.float32) for _ in range(NB)] + [
            pltpu.VMEM((SUB, 128), jnp.float32),
            pltpu.VMEM((128,), jnp.float32),
            pltpu.VMEM_SHARED((NP, 128), jnp.float32),
            pltpu.VMEM_SHARED((NP,), jnp.float32),
            pltpu.VMEM_SHARED((NP,), jnp.float32),
        ] + [pltpu.SemaphoreType.DMA for _ in range(NB + 1)],
    )
    return f(hflat, asrc, adst, sidxf, didxf, brow)


def _edge_phase(hraw, asrc, adst, bias_row, sidxf, didxf):
    return _gat_sc(hraw.reshape(N * CHUNKS, 128), asrc, adst,
                   sidxf, didxf, bias_row)


def kernel(x, edge_index, target, bn_gamma, bn_beta, W1, att_src1, att_dst1,
           b1, W2, att_src2, att_dst2, b2, pool_W, pool_b, dir_W, dir_b):
    src, dst = edge_index[0], edge_index[1]
    loops = jnp.arange(N, dtype=src.dtype)
    pad = jnp.zeros((E_PAD - E_TOT,), src.dtype)
    s_pad = jnp.concatenate([src, loops, pad])
    d_pad = jnp.concatenate([dst, loops, pad])
    sidxf = s_pad.reshape(NS, EPT)
    didxf = d_pad.reshape(NS, EPT)

    W2a, W2b = W2[:D], W2[D:]

    w1as = W1 @ att_src1
    w1ad = W1 @ att_dst1
    w2bs = W2b @ att_src2
    w2bd = W2b @ att_dst2
    extras1 = jnp.zeros((K0, 128), jnp.float32)
    extras1 = extras1.at[:, 0:16].set(dir_W)
    extras1 = extras1.at[:, 16].set(w1as)
    extras1 = extras1.at[:, 17].set(w1ad)
    extras1 = extras1.at[:, 18].set(w2bs)
    extras1 = extras1.at[:, 19].set(w2bd)
    wbig_bf = jnp.concatenate([W1, W2b, extras1], axis=1).astype(jnp.bfloat16)

    w2as = W2a @ att_src2
    w2ad = W2a @ att_dst2
    extras2 = jnp.zeros((D, 128), jnp.float32)
    extras2 = extras2.at[:, 0].set(w2as)
    extras2 = extras2.at[:, 1].set(w2ad)
    w2cat_bf = jnp.concatenate([W2a, extras2], axis=1).astype(jnp.bfloat16)

    pwpad = jnp.zeros((D, 128), jnp.float32).at[:, :C].set(pool_W)
    pwpad_bf = pwpad.astype(jnp.bfloat16)

    sums, sqs = _bn_stats(x)
    mean = sums / N
    var = sqs / N - mean * mean
    scale = bn_gamma / jnp.sqrt(var + 1e-5)
    shift = bn_beta - mean * scale
    scale8 = jnp.broadcast_to(scale[None, :], (8, K0))

    r1 = shift @ W1
    r2 = shift @ W2b

    o1, o2, o3 = _p1(x, scale8, wbig_bf)

    asrc1 = o3[:, 16] + jnp.dot(shift, w1as)
    adst1 = o3[:, 17] + jnp.dot(shift, w1ad)

    h1 = _edge_phase(o1, asrc1, adst1, r1 + b1, sidxf, didxf)

    p2h, p2e = _p2(h1, o2, w2cat_bf)
    asrc2 = o3[:, 18] + p2e[:, 0] + jnp.dot(r2, att_src2)
    adst2 = o3[:, 19] + p2e[:, 1] + jnp.dot(r2, att_dst2)

    h2 = _edge_phase(p2h, asrc2, adst2, r2 + b2, sidxf, didxf)

    cdir = jnp.zeros((128,), jnp.float32).at[:C].set(shift @ dir_W + dir_b)
    cpool = jnp.zeros((128,), jnp.float32).at[:C].set(pool_b)
    cdir8 = jnp.broadcast_to(cdir[None, :], (8, 128))
    cpool8 = jnp.broadcast_to(cpool[None, :], (8, 128))
    tgt_r = target.reshape(NM, 1, BM)

    poolpad, partials = _loss(h2, o3, tgt_r, pwpad_bf, cpool8, cdir8)
    pooler = poolpad[:, :C]
    l = jnp.sum(partials[:, 0, :], axis=0)
    nv = jnp.maximum(l[2], 1.0)
    loss = (l[0] + l[1]) / nv
    acc = l[3] / nv
    return (h2, pooler, loss, acc)


# speedup vs baseline: 1.0950x; 1.0094x over previous
import functools

import jax
import jax.numpy as jnp
import numpy as np
from jax import lax
from jax.experimental import pallas as pl
from jax.experimental.pallas import tpu as pltpu
from jax.experimental.pallas import tpu_sc as plsc

N = 10000
D = 768
C = 16
K0 = 3 * D

BM = 1000
NM = N // BM

NCOL1 = 13 * 128
NCOL2 = 7 * 128


def _stats_body(x_ref, sum_ref, sq_ref):
    @pl.when(pl.program_id(0) == 0)
    def _():
        sum_ref[...] = jnp.zeros_like(sum_ref)
        sq_ref[...] = jnp.zeros_like(sq_ref)
    xb = x_ref[...]
    s = jnp.sum(xb, axis=0, keepdims=True)
    q = jnp.sum(xb * xb, axis=0, keepdims=True)
    sum_ref[...] += jnp.broadcast_to(s, sum_ref.shape)
    sq_ref[...] += jnp.broadcast_to(q, sq_ref.shape)


def _bn_stats(x):
    out = pl.pallas_call(
        _stats_body,
        grid=(NM,),
        in_specs=[pl.BlockSpec((BM, K0), lambda m: (m, 0))],
        out_specs=[pl.BlockSpec((8, K0), lambda m: (0, 0)),
                   pl.BlockSpec((8, K0), lambda m: (0, 0))],
        out_shape=[jax.ShapeDtypeStruct((8, K0), jnp.float32),
                   jax.ShapeDtypeStruct((8, K0), jnp.float32)],
    )(x)
    return out[0][0], out[1][0]


def _p1_body(x_ref, s_ref, w_ref, o1_ref, o2_ref, o3_ref):
    xb = (x_ref[...] * s_ref[0:1, :]).astype(jnp.bfloat16)
    r = jnp.dot(xb, w_ref[...], preferred_element_type=jnp.float32)
    o1_ref[...] = r[:, :768]
    o2_ref[...] = r[:, 768:1536]
    o3_ref[...] = r[:, 1536:]


def _p1(x, scale8, wbig_bf):
    return pl.pallas_call(
        _p1_body,
        grid=(NM,),
        in_specs=[pl.BlockSpec((BM, K0), lambda m: (m, 0)),
                  pl.BlockSpec((8, K0), lambda m: (0, 0)),
                  pl.BlockSpec((K0, NCOL1), lambda m: (0, 0))],
        out_specs=[pl.BlockSpec((BM, 768), lambda m: (m, 0)),
                   pl.BlockSpec((BM, 768), lambda m: (m, 0)),
                   pl.BlockSpec((BM, 128), lambda m: (m, 0))],
        out_shape=[jax.ShapeDtypeStruct((N, 768), jnp.float32),
                   jax.ShapeDtypeStruct((N, 768), jnp.float32),
                   jax.ShapeDtypeStruct((N, 128), jnp.float32)],
    )(x, scale8, wbig_bf)


def _p2_body(h1_ref, xc2_ref, w_ref, oh_ref, oe_ref):
    hb = h1_ref[...].astype(jnp.bfloat16)
    r = jnp.dot(hb, w_ref[...], preferred_element_type=jnp.float32)
    oh_ref[...] = r[:, :768] + xc2_ref[...]
    oe_ref[...] = r[:, 768:]


def _p2(h1, xc2, w2cat_bf):
    return pl.pallas_call(
        _p2_body,
        grid=(NM,),
        in_specs=[pl.BlockSpec((BM, 768), lambda m: (m, 0)),
                  pl.BlockSpec((BM, 768), lambda m: (m, 0)),
                  pl.BlockSpec((768, NCOL2), lambda m: (0, 0))],
        out_specs=[pl.BlockSpec((BM, 768), lambda m: (m, 0)),
                   pl.BlockSpec((BM, 128), lambda m: (m, 0))],
        out_shape=[jax.ShapeDtypeStruct((N, 768), jnp.float32),
                   jax.ShapeDtypeStruct((N, 128), jnp.float32)],
    )(h1, xc2, w2cat_bf)


def _loss_body(h2_ref, o3_ref, tgt_ref, pw_ref, cpool_ref, cdir_ref,
               pool_ref, part_ref):
    h2b = h2_ref[...].astype(jnp.bfloat16)
    pool_logits = jnp.dot(h2b, pw_ref[...], preferred_element_type=jnp.float32)
    pool_logits = pool_logits + cpool_ref[0:1, :]
    dir_logits = o3_ref[...] + cdir_ref[0:1, :]

    lanes = lax.broadcasted_iota(jnp.int32, (BM, 128), 1)
    maskC = lanes < C
    t = tgt_ref[0, 0, :].reshape(BM, 1)
    valid = (t >= 0)
    t_safe = jnp.where(valid, t, 0)
    sel_mask = (lanes == t_safe)

    def ce_rows(L):
        Lm = jnp.where(maskC, L, -jnp.inf)
        m = jnp.max(Lm, axis=1, keepdims=True)
        lse = m + jnp.log(jnp.sum(jnp.where(maskC, jnp.exp(L - m), 0.0),
                                  axis=1, keepdims=True))
        sel = jnp.sum(jnp.where(sel_mask, L, 0.0), axis=1, keepdims=True)
        return lse - sel, Lm, m

    nllp, Lp, mp = ce_rows(pool_logits)
    nlld, _, _ = ce_rows(dir_logits)
    ismax = (Lp == mp) & maskC
    pred = jnp.min(jnp.where(ismax, lanes, 128), axis=1, keepdims=True)
    correct = ((pred == t) & valid).astype(jnp.float32)
    validf = valid.astype(jnp.float32)

    contrib = (nllp * validf * (lanes == 0) + nlld * validf * (lanes == 1)
               + validf * (lanes == 2) + correct * (lanes == 3))
    part = jnp.sum(contrib, axis=0, keepdims=True)
    part_ref[...] = jnp.broadcast_to(part, part_ref.shape)
    pool_ref[...] = pool_logits


def _loss(h2, o3, tgt_r, pwpad_bf, cpool8, cdir8):
    return pl.pallas_call(
        _loss_body,
        grid=(NM,),
        in_specs=[pl.BlockSpec((BM, 768), lambda m: (m, 0)),
                  pl.BlockSpec((BM, 128), lambda m: (m, 0)),
                  pl.BlockSpec((1, 1, BM), lambda m: (m, 0, 0)),
                  pl.BlockSpec((768, 128), lambda m: (0, 0)),
                  pl.BlockSpec((8, 128), lambda m: (0, 0)),
                  pl.BlockSpec((8, 128), lambda m: (0, 0))],
        out_specs=[pl.BlockSpec((BM, 128), lambda m: (m, 0)),
                   pl.BlockSpec((1, 8, 128), lambda m: (m, 0, 0))],
        out_shape=[jax.ShapeDtypeStruct((N, 128), jnp.float32),
                   jax.ShapeDtypeStruct((NM, 8, 128), jnp.float32)],
    )(h2, o3, tgt_r, pwpad_bf, cpool8, cdir8)


E_TOT = 160000 + N
NS = 16
NCORE = 2
G = 16
NG = 672
NB = 8
EPT = NG * G
E_PAD = NS * EPT
NP = 10240
STRIPE = NP // NS
SUB = 40
CHUNKS = 6
CPC = CHUNKS // NCORE
SB = 768
NSB = EPT // SB
GPS = SB // G


def _splat_i32(v):
    return jnp.zeros((16,), jnp.int32) + v


def _gat_sc_body(hflat, asrc, adst, sidxf, didxf, brow, out,
                 sidx_sb, didx_sb, fidx_sb, ad_g, exloc, inv_full,
                 zbuf, b0, b1, b2, b3, b4, b5, b6, b7, wbuf, biasv,
                 acc_sh, den_sh, inv_sh,
                 m0, m1, m2, m3, m4, m5, m6, m7, semP):
    c = lax.axis_index("c")
    s = lax.axis_index("s")
    iota16 = lax.broadcasted_iota(jnp.int32, (16,), 0)
    r0 = s * STRIPE

    pltpu.sync_copy(asrc, inv_full.at[pl.ds(0, N)])

    def _z16(i, carry):
        zbuf[pl.ds(i * 16, 16)] = jnp.zeros((16,), jnp.float32)
        return carry
    lax.fori_loop(0, STRIPE // 16, _z16, None)
    pltpu.sync_copy(zbuf, den_sh.at[pl.ds(r0, STRIPE)])
    plsc.subcore_barrier()

    def _p1(sb, carry):
        base = sb * SB
        pltpu.sync_copy(sidxf.at[s, pl.ds(base, SB)], sidx_sb)
        pltpu.sync_copy(didxf.at[s, pl.ds(base, SB)], didx_sb)
        for q in range(GPS):
            sl = pl.ds(q * G, G)
            pltpu.async_copy(adst.at[didx_sb.at[sl]], ad_g.at[sl], semP)
        for q in range(GPS):
            sl = pl.ds(q * G, G)
            pltpu.make_async_copy(adst.at[didx_sb.at[sl]], ad_g.at[sl], semP).wait()

        def _ex16(i, carry2):
            off = i * 16
            sl = pl.ds(off, 16)
            al = plsc.load_gather(inv_full, [sidx_sb[sl]]) + ad_g[sl]
            al = jnp.where(al >= 0, al, 0.2 * al)
            ex = jnp.exp(al)
            gid = s * EPT + base + off + iota16
            ex = jnp.where(gid < E_TOT, ex, 0.0)
            exloc[pl.ds(base + off, 16)] = ex
            return carry2
        lax.fori_loop(0, SB // 16, _ex16, None)

        for b in range(SB // 16 // 8):
            idxs = [didx_sb[pl.ds((b * 8 + q) * 16, 16)] for q in range(8)]
            for q in range(8):
                pltpu.async_copy(exloc.at[pl.ds(base + (b * 8 + q) * 16, 16)],
                                 den_sh.at[idxs[q]], semP, add=True)
            for q in range(8):
                pltpu.make_async_copy(exloc.at[pl.ds(base + (b * 8 + q) * 16, 16)],
                                      den_sh.at[idxs[q]], semP).wait()
        return carry
    lax.fori_loop(0, NSB, _p1, None)
    plsc.subcore_barrier()

    pltpu.sync_copy(den_sh.at[pl.ds(r0, STRIPE)], zbuf)

    def _inv16(i, carry):
        sl = pl.ds(i * 16, 16)
        zbuf[sl] = 1.0 / (zbuf[sl] + 1e-16)
        return carry
    lax.fori_loop(0, STRIPE // 16, _inv16, None)
    pltpu.sync_copy(zbuf, inv_sh.at[pl.ds(r0, STRIPE)])
    plsc.subcore_barrier()
    pltpu.sync_copy(inv_sh, inv_full)

    def _p1d(sb, carry):
        base = sb * SB
        pltpu.sync_copy(didxf.at[s, pl.ds(base, SB)], didx_sb)

        def _c16(i, carry2):
            off = i * 16
            iv = plsc.load_gather(inv_full, [didx_sb[pl.ds(off, 16)]])
            sl = pl.ds(base + off, 16)
            exloc[sl] = exloc[sl] * iv
            return carry2
        lax.fori_loop(0, SB // 16, _c16, None)
        return carry
    lax.fori_loop(0, NSB, _p1d, None)

    def _gidx(g):
        return hflat.at[fidx_sb.at[pl.ds(g * G, G)]]

    def _scat(buf, g, sem, start):
        ih = didx_sb[pl.ds(g * G, 16)]
        if start:
            pltpu.async_copy(buf, acc_sh.at[ih], sem, add=True)
        else:
            pltpu.make_async_copy(buf, acc_sh.at[ih], sem).wait()

    def _scale(buf, basew):
        for r in range(G):
            cf = plsc.load_gather(exloc, [_splat_i32(basew + r)])
            for l in range(8):
                sl = pl.ds(l * 16, 16)
                buf[r, sl] = buf[r, sl] * cf

    def _chunk(j, carry0):
        jc = c * CPC + j
        pltpu.sync_copy(brow.at[pl.ds(jc * 128, 128)], biasv)
        bvals = [biasv[pl.ds(l * 16, 16)] for l in range(8)]
        for r in range(SUB):
            for l in range(8):
                wbuf[r, pl.ds(l * 16, 16)] = bvals[l]
        for k in range(STRIPE // SUB):
            pltpu.sync_copy(wbuf, acc_sh.at[pl.ds(r0 + k * SUB, SUB)])
        plsc.subcore_barrier()

        def _sb2(sb, carry):
            base = sb * SB
            pltpu.sync_copy(sidxf.at[s, pl.ds(base, SB)], sidx_sb)
            pltpu.sync_copy(didxf.at[s, pl.ds(base, SB)], didx_sb)

            def _fx(i, carry2):
                sl = pl.ds(i * 16, 16)
                fidx_sb[sl] = sidx_sb[sl] * CHUNKS + jc
                return carry2
            lax.fori_loop(0, SB // 16, _fx, None)

            bufs = (b0, b1, b2, b3, b4, b5, b6, b7)
            sems = (m0, m1, m2, m3, m4, m5, m6, m7)
            for q in range(NB - 2):
                pltpu.async_copy(_gidx(q), bufs[q], sems[q])

            def _ring(t, carry2):
                for u in range(NB):
                    g = NB * t + u
                    buf, sm = bufs[u], sems[u]
                    pltpu.make_async_copy(_gidx(g), buf, sm).wait()
                    _scale(buf, base + g * G)
                    _scat(buf, g, sm, True)
                    pbuf = bufs[(u + 6) % NB]
                    psm = sems[(u + 6) % NB]
                    if u < 2:
                        @pl.when(t > 0)
                        def _():
                            _scat(pbuf, g - 2, psm, False)
                        pltpu.async_copy(_gidx(g + 6), pbuf, psm)
                    else:
                        _scat(pbuf, g - 2, psm, False)

                        @pl.when(t < GPS // NB - 1)
                        def _():
                            pltpu.async_copy(_gidx(g + 6), pbuf, psm)
                return carry2
            lax.fori_loop(0, GPS // NB, _ring, None)
            _scat(bufs[(GPS - 2) % NB], GPS - 2, sems[(GPS - 2) % NB], False)
            _scat(bufs[(GPS - 1) % NB], GPS - 1, sems[(GPS - 1) % NB], False)
            return carry
        lax.fori_loop(0, NSB, _sb2, None)
        plsc.subcore_barrier()

        for k in range(STRIPE // SUB):
            rk = r0 + k * SUB

            @pl.when(rk < N)
            def _():
                pltpu.sync_copy(acc_sh.at[pl.ds(rk, SUB)], wbuf)
                pltpu.sync_copy(wbuf, out.at[pl.ds(rk, SUB), pl.ds(jc * 128, 128)])
        plsc.subcore_barrier()
        return carry0
    lax.fori_loop(0, CPC, _chunk, None)


def _gat_sc(hflat, asrc, adst, sidxf, didxf, brow):
    mesh = plsc.VectorSubcoreMesh(core_axis_name="c", subcore_axis_name="s")
    f = pl.kernel(
        _gat_sc_body,
        out_type=jax.ShapeDtypeStruct((N, 768), jnp.float32),
        mesh=mesh,
        name="gat_sc",
        compiler_params=pltpu.CompilerParams(needs_layout_passes=False),
        scratch_types=[
            pltpu.VMEM((SB,), jnp.int32),
            pltpu.VMEM((SB,), jnp.int32),
            pltpu.VMEM((SB,), jnp.int32),
            pltpu.VMEM((SB,), jnp.float32),
            pltpu.VMEM((EPT,), jnp.float32),
            pltpu.VMEM((NP,), jnp.float32),
            pltpu.VMEM((STRIPE,), jnp.float32),
        ] + [pltpu.VMEM((G, 128), jnp.float32) for _ in range(NB)] + [
            pltpu.VMEM((SUB, 128), jnp.float32),
            pltpu.VMEM((128,), jnp.float32),
            pltpu.VMEM_SHARED((NP, 128), jnp.float32),
            pltpu.VMEM_SHARED((NP,), jnp.float32),
            pltpu.VMEM_SHARED((NP,), jnp.float32),
        ] + [pltpu.SemaphoreType.DMA for _ in range(NB + 1)],
    )
    return f(hflat, asrc, adst, sidxf, didxf, brow)


def _edge_phase(hraw, asrc, adst, bias_row, sidxf, didxf):
    return _gat_sc(hraw.reshape(N * CHUNKS, 128), asrc, adst,
                   sidxf, didxf, bias_row)


def kernel(x, edge_index, target, bn_gamma, bn_beta, W1, att_src1, att_dst1,
           b1, W2, att_src2, att_dst2, b2, pool_W, pool_b, dir_W, dir_b):
    src, dst = edge_index[0], edge_index[1]
    loops = jnp.arange(N, dtype=src.dtype)
    pad = jnp.zeros((E_PAD - E_TOT,), src.dtype)
    s_pad = jnp.concatenate([src, loops, pad])
    d_pad = jnp.concatenate([dst, loops, pad])
    sidxf = s_pad.reshape(NS, EPT)
    didxf = d_pad.reshape(NS, EPT)

    W2a, W2b = W2[:D], W2[D:]

    w1as = W1 @ att_src1
    w1ad = W1 @ att_dst1
    w2bs = W2b @ att_src2
    w2bd = W2b @ att_dst2
    extras1 = jnp.zeros((K0, 128), jnp.float32)
    extras1 = extras1.at[:, 0:16].set(dir_W)
    extras1 = extras1.at[:, 16].set(w1as)
    extras1 = extras1.at[:, 17].set(w1ad)
    extras1 = extras1.at[:, 18].set(w2bs)
    extras1 = extras1.at[:, 19].set(w2bd)
    wbig_bf = jnp.concatenate([W1, W2b, extras1], axis=1).astype(jnp.bfloat16)

    w2as = W2a @ att_src2
    w2ad = W2a @ att_dst2
    extras2 = jnp.zeros((D, 128), jnp.float32)
    extras2 = extras2.at[:, 0].set(w2as)
    extras2 = extras2.at[:, 1].set(w2ad)
    w2cat_bf = jnp.concatenate([W2a, extras2], axis=1).astype(jnp.bfloat16)

    pwpad = jnp.zeros((D, 128), jnp.float32).at[:, :C].set(pool_W)
    pwpad_bf = pwpad.astype(jnp.bfloat16)

    sums, sqs = _bn_stats(x)
    mean = sums / N
    var = sqs / N - mean * mean
    scale = bn_gamma / jnp.sqrt(var + 1e-5)
    shift = bn_beta - mean * scale
    scale8 = jnp.broadcast_to(scale[None, :], (8, K0))

    r1 = shift @ W1
    r2 = shift @ W2b

    o1, o2, o3 = _p1(x, scale8, wbig_bf)

    asrc1 = o3[:, 16] + jnp.dot(shift, w1as)
    adst1 = o3[:, 17] + jnp.dot(shift, w1ad)

    h1 = _edge_phase(o1, asrc1, adst1, r1 + b1, sidxf, didxf)

    p2h, p2e = _p2(h1, o2, w2cat_bf)
    asrc2 = o3[:, 18] + p2e[:, 0] + jnp.dot(r2, att_src2)
    adst2 = o3[:, 19] + p2e[:, 1] + jnp.dot(r2, att_dst2)

    h2 = _edge_phase(p2h, asrc2, adst2, r2 + b2, sidxf, didxf)

    cdir = jnp.zeros((128,), jnp.float32).at[:C].set(shift @ dir_W + dir_b)
    cpool = jnp.zeros((128,), jnp.float32).at[:C].set(pool_b)
    cdir8 = jnp.broadcast_to(cdir[None, :], (8, 128))
    cpool8 = jnp.broadcast_to(cpool[None, :], (8, 128))
    tgt_r = target.reshape(NM, 1, BM)

    poolpad, partials = _loss(h2, o3, tgt_r, pwpad_bf, cpool8, cdir8)
    pooler = poolpad[:, :C]
    l = jnp.sum(partials[:, 0, :], axis=0)
    nv = jnp.maximum(l[2], 1.0)
    loss = (l[0] + l[1]) / nv
    acc = l[3] / nv
    return (h2, pooler, loss, acc)


# speedup vs baseline: 1.0956x; 1.0006x over previous
import jax
import jax.numpy as jnp
from jax import lax
from jax.experimental import pallas as pl
from jax.experimental.pallas import tpu as pltpu
from jax.experimental.pallas import tpu_sc as plsc

N = 10000
D = 768
C = 16
K0 = 3 * D

BM = 1000
NM = N // BM

NCOL1 = 13 * 128
NCOL2 = 7 * 128


def _stats_body(x_ref, sum_ref, sq_ref):
    @pl.when(pl.program_id(0) == 0)
    def _():
        sum_ref[...] = jnp.zeros_like(sum_ref)
        sq_ref[...] = jnp.zeros_like(sq_ref)
    xb = x_ref[...]
    s = jnp.sum(xb, axis=0, keepdims=True)
    q = jnp.sum(xb * xb, axis=0, keepdims=True)
    sum_ref[...] += jnp.broadcast_to(s, sum_ref.shape)
    sq_ref[...] += jnp.broadcast_to(q, sq_ref.shape)


def _bn_stats(x):
    out = pl.pallas_call(
        _stats_body,
        grid=(NM,),
        in_specs=[pl.BlockSpec((BM, K0), lambda m: (m, 0))],
        out_specs=[pl.BlockSpec((8, K0), lambda m: (0, 0)),
                   pl.BlockSpec((8, K0), lambda m: (0, 0))],
        out_shape=[jax.ShapeDtypeStruct((8, K0), jnp.float32),
                   jax.ShapeDtypeStruct((8, K0), jnp.float32)],
    )(x)
    return out[0][0], out[1][0]


def _p1_body(x_ref, s_ref, w_ref, o1_ref, o2_ref, o3_ref):
    xb = (x_ref[...] * s_ref[0:1, :]).astype(jnp.bfloat16)
    r = jnp.dot(xb, w_ref[...], preferred_element_type=jnp.float32)
    o1_ref[...] = r[:, :768]
    o2_ref[...] = r[:, 768:1536]
    o3_ref[...] = r[:, 1536:]


def _p1(x, scale8, wbig_bf):
    return pl.pallas_call(
        _p1_body,
        grid=(NM,),
        in_specs=[pl.BlockSpec((BM, K0), lambda m: (m, 0)),
                  pl.BlockSpec((8, K0), lambda m: (0, 0)),
                  pl.BlockSpec((K0, NCOL1), lambda m: (0, 0))],
        out_specs=[pl.BlockSpec((BM, 768), lambda m: (m, 0)),
                   pl.BlockSpec((BM, 768), lambda m: (m, 0)),
                   pl.BlockSpec((BM, 128), lambda m: (m, 0))],
        out_shape=[jax.ShapeDtypeStruct((N, 768), jnp.float32),
                   jax.ShapeDtypeStruct((N, 768), jnp.float32),
                   jax.ShapeDtypeStruct((N, 128), jnp.float32)],
    )(x, scale8, wbig_bf)


def _p2_body(h1_ref, xc2_ref, w_ref, oh_ref, oe_ref):
    hb = h1_ref[...].astype(jnp.bfloat16)
    r = jnp.dot(hb, w_ref[...], preferred_element_type=jnp.float32)
    oh_ref[...] = r[:, :768] + xc2_ref[...]
    oe_ref[...] = r[:, 768:]


def _p2(h1, xc2, w2cat_bf):
    return pl.pallas_call(
        _p2_body,
        grid=(NM,),
        in_specs=[pl.BlockSpec((BM, 768), lambda m: (m, 0)),
                  pl.BlockSpec((BM, 768), lambda m: (m, 0)),
                  pl.BlockSpec((768, NCOL2), lambda m: (0, 0))],
        out_specs=[pl.BlockSpec((BM, 768), lambda m: (m, 0)),
                   pl.BlockSpec((BM, 128), lambda m: (m, 0))],
        out_shape=[jax.ShapeDtypeStruct((N, 768), jnp.float32),
                   jax.ShapeDtypeStruct((N, 128), jnp.float32)],
    )(h1, xc2, w2cat_bf)


def _loss_body(h2_ref, o3_ref, tgt_ref, pw_ref, cpool_ref, cdir_ref,
               pool_ref, part_ref):
    h2b = h2_ref[...].astype(jnp.bfloat16)
    pool_logits = jnp.dot(h2b, pw_ref[...], preferred_element_type=jnp.float32)
    pool_logits = pool_logits + cpool_ref[0:1, :]
    dir_logits = o3_ref[...] + cdir_ref[0:1, :]

    lanes = lax.broadcasted_iota(jnp.int32, (BM, 128), 1)
    maskC = lanes < C
    t = tgt_ref[0, 0, :].reshape(BM, 1)
    valid = (t >= 0)
    t_safe = jnp.where(valid, t, 0)
    sel_mask = (lanes == t_safe)

    def ce_rows(L):
        Lm = jnp.where(maskC, L, -jnp.inf)
        m = jnp.max(Lm, axis=1, keepdims=True)
        lse = m + jnp.log(jnp.sum(jnp.where(maskC, jnp.exp(L - m), 0.0),
                                  axis=1, keepdims=True))
        sel = jnp.sum(jnp.where(sel_mask, L, 0.0), axis=1, keepdims=True)
        return lse - sel, Lm, m

    nllp, Lp, mp = ce_rows(pool_logits)
    nlld, _, _ = ce_rows(dir_logits)
    ismax = (Lp == mp) & maskC
    pred = jnp.min(jnp.where(ismax, lanes, 128), axis=1, keepdims=True)
    correct = ((pred == t) & valid).astype(jnp.float32)
    validf = valid.astype(jnp.float32)

    contrib = (nllp * validf * (lanes == 0) + nlld * validf * (lanes == 1)
               + validf * (lanes == 2) + correct * (lanes == 3))
    part = jnp.sum(contrib, axis=0, keepdims=True)
    part_ref[...] = jnp.broadcast_to(part, part_ref.shape)
    pool_ref[...] = pool_logits


def _loss(h2, o3, tgt_r, pwpad_bf, cpool8, cdir8):
    return pl.pallas_call(
        _loss_body,
        grid=(NM,),
        in_specs=[pl.BlockSpec((BM, 768), lambda m: (m, 0)),
                  pl.BlockSpec((BM, 128), lambda m: (m, 0)),
                  pl.BlockSpec((1, 1, BM), lambda m: (m, 0, 0)),
                  pl.BlockSpec((768, 128), lambda m: (0, 0)),
                  pl.BlockSpec((8, 128), lambda m: (0, 0)),
                  pl.BlockSpec((8, 128), lambda m: (0, 0))],
        out_specs=[pl.BlockSpec((BM, 128), lambda m: (m, 0)),
                   pl.BlockSpec((1, 8, 128), lambda m: (m, 0, 0))],
        out_shape=[jax.ShapeDtypeStruct((N, 128), jnp.float32),
                   jax.ShapeDtypeStruct((NM, 8, 128), jnp.float32)],
    )(h2, o3, tgt_r, pwpad_bf, cpool8, cdir8)


E_TOT = 160000 + N
NS = 16
NCORE = 2
G = 16
NB = 8
EPT = 10752
E_PAD = NS * EPT
NP = 10240
STRIPE = NP // NS
SUB = 40
CHUNKS = 6
CPC = CHUNKS // NCORE
SB = 768
NSB = EPT // SB
GPS = SB // G


def _splat_i32(v):
    return jnp.zeros((16,), jnp.int32) + v


def _gat_sc_body(hflat, asrc, adst, sidxf, didxf, brow, out,
                 sidx_sb, didx_sb, fidx_sb, ad_g, exloc, inv_full,
                 zbuf, b0, b1, b2, b3, b4, b5, b6, b7, wbuf, biasv,
                 acc_sh, den_sh, inv_sh,
                 m0, m1, m2, m3, m4, m5, m6, m7, semP):
    c = lax.axis_index("c")
    s = lax.axis_index("s")
    iota16 = lax.broadcasted_iota(jnp.int32, (16,), 0)
    r0 = s * STRIPE

    pltpu.sync_copy(asrc, inv_full.at[pl.ds(0, N)])

    def _z16(i, carry):
        zbuf[pl.ds(i * 16, 16)] = jnp.zeros((16,), jnp.float32)
        return carry
    lax.fori_loop(0, STRIPE // 16, _z16, None)
    pltpu.sync_copy(zbuf, den_sh.at[pl.ds(r0, STRIPE)])
    plsc.subcore_barrier()

    def _p1(sb, carry):
        base = sb * SB
        pltpu.sync_copy(sidxf.at[s, pl.ds(base, SB)], sidx_sb)
        pltpu.sync_copy(didxf.at[s, pl.ds(base, SB)], didx_sb)
        for q in range(GPS):
            sl = pl.ds(q * G, G)
            pltpu.async_copy(adst.at[didx_sb.at[sl]], ad_g.at[sl], semP)
        for q in range(GPS):
            sl = pl.ds(q * G, G)
            pltpu.make_async_copy(adst.at[didx_sb.at[sl]], ad_g.at[sl], semP).wait()

        def _ex16(i, carry2):
            off = i * 16
            sl = pl.ds(off, 16)
            al = plsc.load_gather(inv_full, [sidx_sb[sl]]) + ad_g[sl]
            al = jnp.where(al >= 0, al, 0.2 * al)
            ex = jnp.exp(al)
            gid = s * EPT + base + off + iota16
            ex = jnp.where(gid < E_TOT, ex, 0.0)
            exloc[pl.ds(base + off, 16)] = ex
            return carry2
        lax.fori_loop(0, SB // 16, _ex16, None)

        for b in range(SB // 16 // 8):
            idxs = [didx_sb[pl.ds((b * 8 + q) * 16, 16)] for q in range(8)]
            for q in range(8):
                pltpu.async_copy(exloc.at[pl.ds(base + (b * 8 + q) * 16, 16)],
                                 den_sh.at[idxs[q]], semP, add=True)
            for q in range(8):
                pltpu.make_async_copy(exloc.at[pl.ds(base + (b * 8 + q) * 16, 16)],
                                      den_sh.at[idxs[q]], semP).wait()
        return carry
    lax.fori_loop(0, NSB, _p1, None)
    plsc.subcore_barrier()

    pltpu.sync_copy(den_sh.at[pl.ds(r0, STRIPE)], zbuf)

    def _inv16(i, carry):
        sl = pl.ds(i * 16, 16)
        zbuf[sl] = 1.0 / (zbuf[sl] + 1e-16)
        return carry
    lax.fori_loop(0, STRIPE // 16, _inv16, None)
    pltpu.sync_copy(zbuf, inv_sh.at[pl.ds(r0, STRIPE)])
    plsc.subcore_barrier()
    pltpu.sync_copy(inv_sh, inv_full)

    def _p1d(sb, carry):
        base = sb * SB
        pltpu.sync_copy(didxf.at[s, pl.ds(base, SB)], didx_sb)

        def _c16(i, carry2):
            off = i * 16
            iv = plsc.load_gather(inv_full, [didx_sb[pl.ds(off, 16)]])
            sl = pl.ds(base + off, 16)
            exloc[sl] = exloc[sl] * iv
            return carry2
        lax.fori_loop(0, SB // 16, _c16, None)
        return carry
    lax.fori_loop(0, NSB, _p1d, None)

    def _gidx(g):
        return hflat.at[fidx_sb.at[pl.ds(g * G, G)]]

    def _scat(buf, g, sem, start):
        ih = didx_sb[pl.ds(g * G, 16)]
        if start:
            pltpu.async_copy(buf, acc_sh.at[ih], sem, add=True)
        else:
            pltpu.make_async_copy(buf, acc_sh.at[ih], sem).wait()

    def _scale(buf, basew):
        for r in range(G):
            cf = plsc.load_gather(exloc, [_splat_i32(basew + r)])
            for l in range(8):
                sl = pl.ds(l * 16, 16)
                buf[r, sl] = buf[r, sl] * cf

    def _chunk(j, carry0):
        jc = c * CPC + j
        pltpu.sync_copy(brow.at[pl.ds(jc * 128, 128)], biasv)
        bvals = [biasv[pl.ds(l * 16, 16)] for l in range(8)]
        for r in range(SUB):
            for l in range(8):
                wbuf[r, pl.ds(l * 16, 16)] = bvals[l]
        for k in range(STRIPE // SUB):
            pltpu.sync_copy(wbuf, acc_sh.at[pl.ds(r0 + k * SUB, SUB)])
        plsc.subcore_barrier()

        def _sb2(sb, carry):
            base = sb * SB
            pltpu.sync_copy(sidxf.at[s, pl.ds(base, SB)], sidx_sb)
            pltpu.sync_copy(didxf.at[s, pl.ds(base, SB)], didx_sb)

            def _fx(i, carry2):
                sl = pl.ds(i * 16, 16)
                fidx_sb[sl] = sidx_sb[sl] * CHUNKS + jc
                return carry2
            lax.fori_loop(0, SB // 16, _fx, None)

            bufs = (b0, b1, b2, b3, b4, b5, b6, b7)
            sems = (m0, m1, m2, m3, m4, m5, m6, m7)
            for q in range(NB - 2):
                pltpu.async_copy(_gidx(q), bufs[q], sems[q])

            def _ring(t, carry2):
                for u in range(NB):
                    g = NB * t + u
                    buf, sm = bufs[u], sems[u]
                    pltpu.make_async_copy(_gidx(g), buf, sm).wait()
                    _scale(buf, base + g * G)
                    _scat(buf, g, sm, True)
                    pbuf = bufs[(u + 6) % NB]
                    psm = sems[(u + 6) % NB]
                    if u < 2:
                        @pl.when(t > 0)
                        def _():
                            _scat(pbuf, g - 2, psm, False)
                        pltpu.async_copy(_gidx(g + 6), pbuf, psm)
                    else:
                        _scat(pbuf, g - 2, psm, False)

                        @pl.when(t < GPS // NB - 1)
                        def _():
                            pltpu.async_copy(_gidx(g + 6), pbuf, psm)
                return carry2
            lax.fori_loop(0, GPS // NB, _ring, None)
            _scat(bufs[(GPS - 2) % NB], GPS - 2, sems[(GPS - 2) % NB], False)
            _scat(bufs[(GPS - 1) % NB], GPS - 1, sems[(GPS - 1) % NB], False)
            return carry
        lax.fori_loop(0, NSB, _sb2, None)
        plsc.subcore_barrier()

        for k in range(STRIPE // SUB):
            rk = r0 + k * SUB

            @pl.when(rk < N)
            def _():
                pltpu.sync_copy(acc_sh.at[pl.ds(rk, SUB)], wbuf)
                pltpu.sync_copy(wbuf, out.at[pl.ds(rk, SUB), pl.ds(jc * 128, 128)])
        plsc.subcore_barrier()
        return carry0
    lax.fori_loop(0, CPC, _chunk, None)


def _gat_sc(hflat, asrc, adst, sidxf, didxf, brow):
    mesh = plsc.VectorSubcoreMesh(core_axis_name="c", subcore_axis_name="s")
    f = pl.kernel(
        _gat_sc_body,
        out_type=jax.ShapeDtypeStruct((N, 768), jnp.float32),
        mesh=mesh,
        name="gat_sc",
        compiler_params=pltpu.CompilerParams(needs_layout_passes=False),
        scratch_types=[
            pltpu.VMEM((SB,), jnp.int32),
            pltpu.VMEM((SB,), jnp.int32),
            pltpu.VMEM((SB,), jnp.int32),
            pltpu.VMEM((SB,), jnp.float32),
            pltpu.VMEM((EPT,), jnp.float32),
            pltpu.VMEM((NP,), jnp.float32),
            pltpu.VMEM((STRIPE,), jnp.float32),
        ] + [pltpu.VMEM((G, 128), jnp.float32) for _ in range(NB)] + [
            pltpu.VMEM((SUB, 128), jnp.float32),
            pltpu.VMEM((128,), jnp.float32),
            pltpu.VMEM_SHARED((NP, 128), jnp.float32),
            pltpu.VMEM_SHARED((NP,), jnp.float32),
            pltpu.VMEM_SHARED((NP,), jnp.float32),
        ] + [pltpu.SemaphoreType.DMA for _ in range(NB + 1)],
    )
    return f(hflat, asrc, adst, sidxf, didxf, brow)


def _edge_phase(hraw, asrc, adst, bias_row, sidxf, didxf):
    return _gat_sc(hraw.reshape(N * CHUNKS, 128), asrc, adst,
                   sidxf, didxf, bias_row)


def kernel(x, edge_index, target, bn_gamma, bn_beta, W1, att_src1, att_dst1,
           b1, W2, att_src2, att_dst2, b2, pool_W, pool_b, dir_W, dir_b):
    src, dst = edge_index[0], edge_index[1]
    loops = jnp.arange(N, dtype=src.dtype)
    pad = jnp.zeros((E_PAD - E_TOT,), src.dtype)
    s_pad = jnp.concatenate([src, loops, pad])
    d_pad = jnp.concatenate([dst, loops, pad])
    sidxf = s_pad.reshape(NS, EPT)
    didxf = d_pad.reshape(NS, EPT)

    W2a, W2b = W2[:D], W2[D:]

    w1as = W1 @ att_src1
    w1ad = W1 @ att_dst1
    w2bs = W2b @ att_src2
    w2bd = W2b @ att_dst2
    extras1 = jnp.zeros((K0, 128), jnp.float32)
    extras1 = extras1.at[:, 0:16].set(dir_W)
    extras1 = extras1.at[:, 16].set(w1as)
    extras1 = extras1.at[:, 17].set(w1ad)
    extras1 = extras1.at[:, 18].set(w2bs)
    extras1 = extras1.at[:, 19].set(w2bd)
    wbig_bf = jnp.concatenate([W1, W2b, extras1], axis=1).astype(jnp.bfloat16)

    w2as = W2a @ att_src2
    w2ad = W2a @ att_dst2
    extras2 = jnp.zeros((D, 128), jnp.float32)
    extras2 = extras2.at[:, 0].set(w2as)
    extras2 = extras2.at[:, 1].set(w2ad)
    w2cat_bf = jnp.concatenate([W2a, extras2], axis=1).astype(jnp.bfloat16)

    pwpad = jnp.zeros((D, 128), jnp.float32).at[:, :C].set(pool_W)
    pwpad_bf = pwpad.astype(jnp.bfloat16)

    sums, sqs = _bn_stats(x)
    mean = sums / N
    var = sqs / N - mean * mean
    scale = bn_gamma / jnp.sqrt(var + 1e-5)
    shift = bn_beta - mean * scale
    scale8 = jnp.broadcast_to(scale[None, :], (8, K0))

    r1 = shift @ W1
    r2 = shift @ W2b

    o1, o2, o3 = _p1(x, scale8, wbig_bf)

    asrc1 = o3[:, 16] + jnp.dot(shift, w1as)
    adst1 = o3[:, 17] + jnp.dot(shift, w1ad)

    h1 = _edge_phase(o1, asrc1, adst1, r1 + b1, sidxf, didxf)

    p2h, p2e = _p2(h1, o2, w2cat_bf)
    asrc2 = o3[:, 18] + p2e[:, 0] + jnp.dot(r2, att_src2)
    adst2 = o3[:, 19] + p2e[:, 1] + jnp.dot(r2, att_dst2)

    h2 = _edge_phase(p2h, asrc2, adst2, r2 + b2, sidxf, didxf)

    cdir = jnp.zeros((128,), jnp.float32).at[:C].set(shift @ dir_W + dir_b)
    cpool = jnp.zeros((128,), jnp.float32).at[:C].set(pool_b)
    cdir8 = jnp.broadcast_to(cdir[None, :], (8, 128))
    cpool8 = jnp.broadcast_to(cpool[None, :], (8, 128))
    tgt_r = target.reshape(NM, 1, BM)

    poolpad, partials = _loss(h2, o3, tgt_r, pwpad_bf, cpool8, cdir8)
    pooler = poolpad[:, :C]
    l = jnp.sum(partials[:, 0, :], axis=0)
    nv = jnp.maximum(l[2], 1.0)
    loss = (l[0] + l[1]) / nv
    acc = l[3] / nv
    return (h2, pooler, loss, acc)
